# Initial kernel scaffold; baseline (speedup 1.0000x reference)
import functools

import jax
import jax.numpy as jnp
from jax import lax
from jax.experimental import pallas as pl
from jax.experimental.pallas import tpu as pltpu
from jax.experimental.pallas import tpu_sc as plsc

B = 16384
L = 200
D = 64
NCLS = 1000

_NC = 2
_NS = 16
NW = _NC * _NS
BPW = B // NW
HALF = BPW // 2
LH = L // 2
NVREG = D // 16


def _sc_embed_sum(idx2, table):
    mesh = plsc.VectorSubcoreMesh(core_axis_name="c", subcore_axis_name="s")

    @functools.partial(
        pl.kernel,
        mesh=mesh,
        out_type=jax.ShapeDtypeStruct((B, D), jnp.float32),
        scratch_types=[
            pltpu.VMEM((2 * HALF, LH), jnp.int32),
            pltpu.VMEM((2, 2, LH, D), jnp.float32),
            pltpu.VMEM((HALF, D), jnp.float32),
            pltpu.SemaphoreType.DMA,
            pltpu.SemaphoreType.DMA,
        ],
    )
    def run(idx_hbm, tab_hbm, out_hbm, idx_v, rows_v, out_v, sem0, sem1):
        c = lax.axis_index("c")
        s = lax.axis_index("s")
        wid = s * _NC + c
        row0 = wid * BPW

        def sum_row(buf, r):
            def body(i, accs):
                out = []
                for q in range(NVREG):
                    a = accs[q]
                    for j in range(2):
                        a = a + rows_v[buf, j, i, pl.ds(q * 16, 16)]
                    out.append(a)
                return tuple(out)

            accs = lax.fori_loop(
                0, LH, body,
                tuple(jnp.zeros((16,), jnp.float32) for _ in range(NVREG)))
            for q in range(NVREG):
                out_v[r, pl.ds(q * 16, 16)] = accs[q]

        for h in range(2):
            hrow = row0 + h * HALF
            pltpu.sync_copy(idx_hbm.at[pl.ds(2 * hrow, 2 * HALF)], idx_v)

            def pair(p, _):
                cp0 = pltpu.async_copy(
                    tab_hbm.at[idx_v.at[pl.ds(4 * p, 2)]], rows_v.at[0], sem0)
                cp1 = pltpu.async_copy(
                    tab_hbm.at[idx_v.at[pl.ds(4 * p + 2, 2)]], rows_v.at[1],
                    sem1)
                cp0.wait()
                sum_row(0, 2 * p)
                cp1.wait()
                sum_row(1, 2 * p + 1)
                return 0

            lax.fori_loop(0, HALF // 2, pair, 0)
            pltpu.sync_copy(out_v, out_hbm.at[pl.ds(hrow, HALF)])

    return run(idx2, table)


_BM = 1024


def _tc_project(emb, bias2, w, pb2):
    def body(x_ref, b_ref, w_ref, pb_ref, o_ref):
        x = x_ref[...] + b_ref[...]
        o_ref[...] = (
            jnp.dot(x, w_ref[...], preferred_element_type=jnp.float32)
            + pb_ref[...])

    return pl.pallas_call(
        body,
        grid=(B // _BM,),
        in_specs=[
            pl.BlockSpec((_BM, D), lambda i: (i, 0)),
            pl.BlockSpec((1, D), lambda i: (0, 0)),
            pl.BlockSpec((D, NCLS), lambda i: (0, 0)),
            pl.BlockSpec((1, NCLS), lambda i: (0, 0)),
        ],
        out_specs=pl.BlockSpec((_BM, NCLS), lambda i: (i, 0)),
        out_shape=jax.ShapeDtypeStruct((B, NCLS), jnp.float32),
    )(emb, bias2, w, pb2)


def kernel(inputs, embed_table, bias, proj_w, proj_b):
    idx2 = inputs.reshape(2 * B, LH)
    emb = _sc_embed_sum(idx2, embed_table)
    return _tc_project(emb, bias.reshape(1, D), proj_w,
                       proj_b.reshape(1, NCLS))


# speedup vs baseline: 2.3574x; 2.3574x over previous
import functools

import jax
import jax.numpy as jnp
from jax import lax
from jax.experimental import pallas as pl
from jax.experimental.pallas import tpu as pltpu
from jax.experimental.pallas import tpu_sc as plsc

B = 16384
L = 200
D = 64
NCLS = 1000

_NC = 2
_NS = 16
NW = _NC * _NS
BPW = B // NW
HALF = BPW // 2
LH = L // 2
NVREG = D // 16


def _sc_embed_sum(idx2, table):
    mesh = plsc.VectorSubcoreMesh(core_axis_name="c", subcore_axis_name="s")

    @functools.partial(
        pl.kernel,
        mesh=mesh,
        out_type=jax.ShapeDtypeStruct((B, D), jnp.float32),
        scratch_types=[
            pltpu.VMEM((2 * HALF, LH), jnp.int32),
            pltpu.VMEM((2, 2, LH, D), jnp.float32),
            pltpu.VMEM((HALF, D), jnp.float32),
            pltpu.SemaphoreType.DMA,
            pltpu.SemaphoreType.DMA,
        ],
        compiler_params=pltpu.CompilerParams(use_tc_tiling_on_sc=False),
    )
    def run(idx_hbm, tab_hbm, out_hbm, idx_v, rows_v, out_v, sem0, sem1):
        c = lax.axis_index("c")
        s = lax.axis_index("s")
        wid = s * _NC + c
        row0 = wid * BPW

        def sum_row(buf, r):
            def body(i, accs):
                out = []
                for q in range(NVREG):
                    a = accs[q]
                    for j in range(2):
                        a = a + rows_v[buf, j, i, pl.ds(q * 16, 16)]
                    out.append(a)
                return tuple(out)

            accs = lax.fori_loop(
                0, LH, body,
                tuple(jnp.zeros((16,), jnp.float32) for _ in range(NVREG)))
            for q in range(NVREG):
                out_v[r, pl.ds(q * 16, 16)] = accs[q]

        for h in range(2):
            hrow = row0 + h * HALF
            pltpu.sync_copy(idx_hbm.at[pl.ds(2 * hrow, 2 * HALF)], idx_v)

            def pair(p, _):
                cp0a = pltpu.async_copy(
                    tab_hbm.at[idx_v.at[4 * p]], rows_v.at[0, 0], sem0)
                cp0b = pltpu.async_copy(
                    tab_hbm.at[idx_v.at[4 * p + 1]], rows_v.at[0, 1], sem0)
                cp1a = pltpu.async_copy(
                    tab_hbm.at[idx_v.at[4 * p + 2]], rows_v.at[1, 0], sem1)
                cp1b = pltpu.async_copy(
                    tab_hbm.at[idx_v.at[4 * p + 3]], rows_v.at[1, 1], sem1)
                cp0a.wait()
                cp0b.wait()
                sum_row(0, 2 * p)
                cp1a.wait()
                cp1b.wait()
                sum_row(1, 2 * p + 1)
                return 0

            lax.fori_loop(0, HALF // 2, pair, 0)
            pltpu.sync_copy(out_v, out_hbm.at[pl.ds(hrow, HALF)])

    return run(idx2, table)


_BM = 1024


def _tc_project(emb, bias2, w, pb2):
    def body(x_ref, b_ref, w_ref, pb_ref, o_ref):
        x = x_ref[...] + b_ref[...]
        o_ref[...] = (
            jnp.dot(x, w_ref[...], preferred_element_type=jnp.float32)
            + pb_ref[...])

    return pl.pallas_call(
        body,
        grid=(B // _BM,),
        in_specs=[
            pl.BlockSpec((_BM, D), lambda i: (i, 0)),
            pl.BlockSpec((1, D), lambda i: (0, 0)),
            pl.BlockSpec((D, NCLS), lambda i: (0, 0)),
            pl.BlockSpec((1, NCLS), lambda i: (0, 0)),
        ],
        out_specs=pl.BlockSpec((_BM, NCLS), lambda i: (i, 0)),
        out_shape=jax.ShapeDtypeStruct((B, NCLS), jnp.float32),
    )(emb, bias2, w, pb2)


def kernel(inputs, embed_table, bias, proj_w, proj_b):
    idx2 = inputs.reshape(2 * B, LH)
    emb = _sc_embed_sum(idx2, embed_table)
    return _tc_project(emb, bias.reshape(1, D), proj_w,
                       proj_b.reshape(1, NCLS))


# speedup vs baseline: 3.3789x; 1.4333x over previous
import functools

import jax
import jax.numpy as jnp
from jax import lax
from jax.experimental import pallas as pl
from jax.experimental.pallas import tpu as pltpu
from jax.experimental.pallas import tpu_sc as plsc

B = 16384
L = 200
D = 64
NCLS = 1000

_NC = 2
_NS = 16
NW = _NC * _NS
BPW = B // NW
HALF = BPW // 2
LH = L // 2
NVREG = D // 16
NSLOT = 4


def _sc_embed_sum(idx2, table):
    mesh = plsc.VectorSubcoreMesh(core_axis_name="c", subcore_axis_name="s")

    @functools.partial(
        pl.kernel,
        mesh=mesh,
        out_type=jax.ShapeDtypeStruct((B, D), jnp.float32),
        scratch_types=[
            pltpu.VMEM((2 * HALF, LH), jnp.int32),
            pltpu.VMEM((NSLOT, 2, LH, D), jnp.float32),
            pltpu.VMEM((HALF, D), jnp.float32),
            [pltpu.SemaphoreType.DMA] * NSLOT,
        ],
        compiler_params=pltpu.CompilerParams(use_tc_tiling_on_sc=False),
    )
    def run(idx_hbm, tab_hbm, out_hbm, idx_v, rows_v, out_v, sems):
        c = lax.axis_index("c")
        s = lax.axis_index("s")
        wid = s * _NC + c
        row0 = wid * BPW

        def issue(slot, r):
            pltpu.async_copy(
                tab_hbm.at[idx_v.at[2 * r]], rows_v.at[slot, 0], sems[slot])
            pltpu.async_copy(
                tab_hbm.at[idx_v.at[2 * r + 1]], rows_v.at[slot, 1],
                sems[slot])

        def wait(slot):
            pltpu.make_async_copy(
                tab_hbm.at[idx_v.at[0]], rows_v.at[slot, 0],
                sems[slot]).wait()
            pltpu.make_async_copy(
                tab_hbm.at[idx_v.at[1]], rows_v.at[slot, 1],
                sems[slot]).wait()

        def sum_row(slot, r):
            def body(i, accs):
                out = []
                for q in range(NVREG):
                    a = accs[q]
                    for j in range(2):
                        a = a + rows_v[slot, j, i, pl.ds(q * 16, 16)]
                    out.append(a)
                return tuple(out)

            accs = lax.fori_loop(
                0, LH, body,
                tuple(jnp.zeros((16,), jnp.float32) for _ in range(NVREG)),
                unroll=4)
            for q in range(NVREG):
                out_v[r, pl.ds(q * 16, 16)] = accs[q]

        for h in range(2):
            hrow = row0 + h * HALF
            pltpu.sync_copy(idx_hbm.at[pl.ds(2 * hrow, 2 * HALF)], idx_v)

            for k in range(NSLOT):
                issue(k, k)

            def group(g, _):
                for k in range(NSLOT):
                    r = g * NSLOT + k
                    wait(k)
                    sum_row(k, r)
                    nr = r + NSLOT

                    @pl.when(nr < HALF)
                    def _():
                        issue(k, nr)

                return 0

            lax.fori_loop(0, HALF // NSLOT, group, 0)
            pltpu.sync_copy(out_v, out_hbm.at[pl.ds(hrow, HALF)])

    return run(idx2, table)


_BM = 1024


def _tc_project_t(emb, bias2, w, pbt):
    def body(x_ref, b_ref, w_ref, pb_ref, o_ref):
        x = x_ref[...] + b_ref[...]
        o_ref[...] = (
            lax.dot_general(w_ref[...], x,
                            (((0,), (1,)), ((), ())),
                            preferred_element_type=jnp.float32)
            + pb_ref[...])

    return pl.pallas_call(
        body,
        grid=(B // _BM,),
        in_specs=[
            pl.BlockSpec((_BM, D), lambda i: (i, 0)),
            pl.BlockSpec((1, D), lambda i: (0, 0)),
            pl.BlockSpec((D, NCLS), lambda i: (0, 0)),
            pl.BlockSpec((NCLS, 1), lambda i: (0, 0)),
        ],
        out_specs=pl.BlockSpec((NCLS, _BM), lambda i: (0, i)),
        out_shape=jax.ShapeDtypeStruct((NCLS, B), jnp.float32),
    )(emb, bias2, w, pbt)


def kernel(inputs, embed_table, bias, proj_w, proj_b):
    idx2 = inputs.reshape(2 * B, LH)
    emb = _sc_embed_sum(idx2, embed_table)
    logits_t = _tc_project_t(emb, bias.reshape(1, D), proj_w,
                             proj_b.reshape(NCLS, 1))
    return logits_t.T
